# Initial kernel scaffold; baseline (speedup 1.0000x reference)
import jax
import jax.numpy as jnp
from jax.experimental import pallas as pl


def kernel(x, edge_index, W1l, b1l, W1r, W2l, b2l, W2r):
    raise NotImplementedError("write your pallas kernel here")


# speedup vs baseline: 5.9125x; 5.9125x over previous
import functools

import jax
import jax.numpy as jnp
from jax import lax
from jax.experimental import pallas as pl
from jax.experimental.pallas import tpu as pltpu
from jax.experimental.pallas import tpu_sc as plsc

N_NODES = 10000
N_EDGES = 320000
D_FEAT = 128
HIDDEN = 128
N_CLASSES = 16
AUG = 16
D_A = HIDDEN + AUG

NC = 2
NS = 16
NW = NC * NS
EPW = N_EDGES // NW
CH = 80
NCHUNK = EPW // CH
N_PAD = 10240
ROWS_PER_TILE = N_PAD // NS
ZROWS = 128

BLK = 1000


@functools.lru_cache(maxsize=None)
def _make_sc_scatter_pass(d):
  mesh = plsc.VectorSubcoreMesh(
      core_axis_name="c", subcore_axis_name="s",
      num_cores=NC, num_subcores=NS)

  @functools.partial(
      pl.kernel,
      out_type=jax.ShapeDtypeStruct((NC, N_PAD, d), jnp.float32),
      mesh=mesh,
      compiler_params=pltpu.CompilerParams(use_tc_tiling_on_sc=False),
      scratch_types=[
          pltpu.VMEM((CH,), jnp.int32),
          pltpu.VMEM((CH,), jnp.int32),
          pltpu.VMEM((CH, d), jnp.float32),
          pltpu.VMEM((ZROWS, d), jnp.float32),
          pltpu.VMEM_SHARED((N_PAD, d), jnp.float32),
          pltpu.SemaphoreType.DMA,
      ],
  )
  def scatter_pass(vals_hbm, src_hbm, dst_hbm, out_hbm,
                   src_v, dst_v, rows_v, zero_v, acc_sh, sem):
    cid = lax.axis_index("c")
    sid = lax.axis_index("s")
    wid = sid * NC + cid

    zvec = jnp.zeros((16,), jnp.float32)

    def zero_row(i, carry):
      for j in range(d // 16):
        zero_v[i, pl.ds(j * 16, 16)] = zvec
      return carry

    lax.fori_loop(0, ZROWS, zero_row, 0)
    for k in range(ROWS_PER_TILE // ZROWS):
      pltpu.sync_copy(
          zero_v,
          acc_sh.at[pl.ds(sid * ROWS_PER_TILE + k * ZROWS, ZROWS), :])
    plsc.subcore_barrier()

    base0 = wid * EPW

    def body(i, carry):
      base = base0 + i * CH
      pltpu.sync_copy(src_hbm.at[pl.ds(base, CH)], src_v)
      pltpu.sync_copy(dst_hbm.at[pl.ds(base, CH)], dst_v)
      pltpu.async_copy(vals_hbm.at[src_v], rows_v, sem).wait()
      pltpu.sync_copy(rows_v, acc_sh.at[dst_v], add=True)
      return carry

    lax.fori_loop(0, NCHUNK, body, 0)
    plsc.subcore_barrier()

    pltpu.sync_copy(
        acc_sh.at[pl.ds(sid * ROWS_PER_TILE, ROWS_PER_TILE), :],
        out_hbm.at[cid, pl.ds(sid * ROWS_PER_TILE, ROWS_PER_TILE), :])

  return scatter_pass


def _dot_t(a, b):
  return lax.dot_general(a, b, (((1,), (1,)), ((), ())),
                         preferred_element_type=jnp.float32)


def _tc1_body(x_ref, w1l_ref, o_ref):
  o_ref[:, :HIDDEN] = _dot_t(x_ref[...], w1l_ref[...])
  o_ref[:, HIDDEN:] = jnp.ones((BLK, AUG), jnp.float32)


def _tc1(x, w1l):
  return pl.pallas_call(
      _tc1_body,
      grid=(N_NODES // BLK,),
      in_specs=[
          pl.BlockSpec((BLK, D_FEAT), lambda i: (i, 0)),
          pl.BlockSpec((HIDDEN, D_FEAT), lambda i: (0, 0)),
      ],
      out_specs=pl.BlockSpec((BLK, D_A), lambda i: (i, 0)),
      out_shape=jax.ShapeDtypeStruct((N_NODES, D_A), jnp.float32),
  )(x, w1l)


def _tc2_body(p_ref, x_ref, w1r_ref, b1l_ref, w2l_ref, b2l_ref, w2r_ref,
              t2_ref, hr_ref, invd_ref):
  s = p_ref[0] + p_ref[1]
  deg = jnp.maximum(s[:, HIDDEN:HIDDEN + 1], 1.0)
  invd = 1.0 / deg
  agg = s[:, :HIDDEN] * invd
  h = agg + b1l_ref[...] + _dot_t(x_ref[...], w1r_ref[...])
  h = jnp.maximum(h, 0.0)
  t2_ref[...] = _dot_t(h, w2l_ref[...])
  hr_ref[...] = _dot_t(h, w2r_ref[...]) + b2l_ref[...]
  invd_ref[...] = jnp.broadcast_to(invd, (BLK, N_CLASSES))


def _tc2(p, x, w1r, b1l2, w2l, b2l2, w2r):
  return pl.pallas_call(
      _tc2_body,
      grid=(N_NODES // BLK,),
      in_specs=[
          pl.BlockSpec((NC, BLK, D_A), lambda i: (0, i, 0)),
          pl.BlockSpec((BLK, D_FEAT), lambda i: (i, 0)),
          pl.BlockSpec((HIDDEN, D_FEAT), lambda i: (0, 0)),
          pl.BlockSpec((1, HIDDEN), lambda i: (0, 0)),
          pl.BlockSpec((N_CLASSES, HIDDEN), lambda i: (0, 0)),
          pl.BlockSpec((1, N_CLASSES), lambda i: (0, 0)),
          pl.BlockSpec((N_CLASSES, HIDDEN), lambda i: (0, 0)),
      ],
      out_specs=[
          pl.BlockSpec((BLK, N_CLASSES), lambda i: (i, 0)),
          pl.BlockSpec((BLK, N_CLASSES), lambda i: (i, 0)),
          pl.BlockSpec((BLK, N_CLASSES), lambda i: (i, 0)),
      ],
      out_shape=[
          jax.ShapeDtypeStruct((N_NODES, N_CLASSES), jnp.float32),
          jax.ShapeDtypeStruct((N_NODES, N_CLASSES), jnp.float32),
          jax.ShapeDtypeStruct((N_NODES, N_CLASSES), jnp.float32),
      ],
  )(p, x, w1r, b1l2, w2l, b2l2, w2r)


def _tc3_body(q_ref, invd_ref, hr_ref, o_ref):
  o_ref[...] = (q_ref[0] + q_ref[1]) * invd_ref[...] + hr_ref[...]


def _tc3(q, invd, hr):
  return pl.pallas_call(
      _tc3_body,
      grid=(N_NODES // BLK,),
      in_specs=[
          pl.BlockSpec((NC, BLK, N_CLASSES), lambda i: (0, i, 0)),
          pl.BlockSpec((BLK, N_CLASSES), lambda i: (i, 0)),
          pl.BlockSpec((BLK, N_CLASSES), lambda i: (i, 0)),
      ],
      out_specs=pl.BlockSpec((BLK, N_CLASSES), lambda i: (i, 0)),
      out_shape=jax.ShapeDtypeStruct((N_NODES, N_CLASSES), jnp.float32),
  )(q, invd, hr)


def kernel(x, edge_index, W1l, b1l, W1r, W2l, b2l, W2r):
  src = edge_index[0].astype(jnp.int32)
  dst = edge_index[1].astype(jnp.int32)
  b1l2 = b1l.reshape(1, HIDDEN)
  b2l2 = b2l.reshape(1, N_CLASSES)

  t1aug = _tc1(x, W1l)
  p = _make_sc_scatter_pass(D_A)(t1aug, src, dst)
  t2, hr, invd = _tc2(p, x, W1r, b1l2, W2l, b2l2, W2r)
  q = _make_sc_scatter_pass(N_CLASSES)(t2, src, dst)
  return _tc3(q, invd, hr)


# speedup vs baseline: 15.0964x; 2.5533x over previous
import functools

import jax
import jax.numpy as jnp
from jax import lax
from jax.experimental import pallas as pl
from jax.experimental.pallas import tpu as pltpu
from jax.experimental.pallas import tpu_sc as plsc

N_NODES = 10000
N_EDGES = 320000
D_FEAT = 128
HIDDEN = 128
N_CLASSES = 16
AUG = 16
HALF = 64
D_A1 = HALF + AUG
D_A2 = HALF

NC = 2
NS = 16
NW = NC * NS
EPW = N_EDGES // NW
CH = 80
NCHUNK = EPW // CH
N_PAD = 10240
ROWS_PER_TILE = N_PAD // NS
ZROWS = 64
NBUF = 6
KPF = 4

BLK = 1000


@functools.lru_cache(maxsize=None)
def _make_sc_scatter_pass(d):
  mesh = plsc.VectorSubcoreMesh(
      core_axis_name="c", subcore_axis_name="s",
      num_cores=NC, num_subcores=NS)

  @functools.partial(
      pl.kernel,
      out_type=jax.ShapeDtypeStruct((NC, N_PAD, d), jnp.float32),
      mesh=mesh,
      compiler_params=pltpu.CompilerParams(use_tc_tiling_on_sc=False),
      scratch_types=[
          pltpu.VMEM((NCHUNK, CH), jnp.int32),
          pltpu.VMEM((NCHUNK, CH), jnp.int32),
          pltpu.VMEM((NBUF, CH, d), jnp.float32),
          pltpu.VMEM((ZROWS, d), jnp.float32),
          pltpu.VMEM_SHARED((N_PAD, d), jnp.float32),
          pltpu.SemaphoreType.DMA,
      ],
  )
  def scatter_pass(vals_hbm, src_hbm, dst_hbm, out_hbm,
                   src_v, dst_v, rows_v, zero_v, acc_sh, gsem):
    cid = lax.axis_index("c")
    sid = lax.axis_index("s")
    wid = sid * NC + cid

    pltpu.sync_copy(src_hbm.at[pl.ds(wid * NCHUNK, NCHUNK), :], src_v)
    pltpu.sync_copy(dst_hbm.at[pl.ds(wid * NCHUNK, NCHUNK), :], dst_v)

    zvec = jnp.zeros((16,), jnp.float32)

    def zero_row(i, carry):
      for j in range(d // 16):
        zero_v[i, pl.ds(j * 16, 16)] = zvec
      return carry

    lax.fori_loop(0, ZROWS, zero_row, 0)
    for k in range(ROWS_PER_TILE // ZROWS):
      pltpu.sync_copy(
          zero_v,
          acc_sh.at[pl.ds(sid * ROWS_PER_TILE + k * ZROWS, ZROWS), :])
    plsc.subcore_barrier()

    for k in range(KPF):
      pltpu.async_copy(vals_hbm.at[src_v.at[k]], rows_v.at[k], gsem)

    def body(i, carry):
      @pl.when(i + KPF < NCHUNK)
      def _():
        pltpu.async_copy(vals_hbm.at[src_v.at[i + KPF]],
                         rows_v.at[lax.rem(i + KPF, NBUF)], gsem)
      b = lax.rem(i, NBUF)
      pltpu.make_async_copy(vals_hbm.at[src_v.at[i]], rows_v.at[b],
                            gsem).wait()
      pltpu.sync_copy(rows_v.at[b], acc_sh.at[dst_v.at[i]], add=True)
      return carry

    lax.fori_loop(0, NCHUNK, body, 0)
    plsc.subcore_barrier()

    pltpu.sync_copy(
        acc_sh.at[pl.ds(sid * ROWS_PER_TILE, ROWS_PER_TILE), :],
        out_hbm.at[cid, pl.ds(sid * ROWS_PER_TILE, ROWS_PER_TILE), :])

  return scatter_pass


def _dot_t(a, b):
  return lax.dot_general(a, b, (((1,), (1,)), ((), ())),
                         preferred_element_type=jnp.float32)


def _tc1_body(x_ref, w1l_ref, oa_ref, ob_ref):
  t = _dot_t(x_ref[...], w1l_ref[...])
  oa_ref[:, :HALF] = t[:, :HALF]
  oa_ref[:, HALF:] = jnp.ones((BLK, AUG), jnp.float32)
  ob_ref[...] = t[:, HALF:]


def _tc1(x, w1l):
  return pl.pallas_call(
      _tc1_body,
      grid=(N_NODES // BLK,),
      in_specs=[
          pl.BlockSpec((BLK, D_FEAT), lambda i: (i, 0)),
          pl.BlockSpec((HIDDEN, D_FEAT), lambda i: (0, 0)),
      ],
      out_specs=[
          pl.BlockSpec((BLK, D_A1), lambda i: (i, 0)),
          pl.BlockSpec((BLK, D_A2), lambda i: (i, 0)),
      ],
      out_shape=[
          jax.ShapeDtypeStruct((N_NODES, D_A1), jnp.float32),
          jax.ShapeDtypeStruct((N_NODES, D_A2), jnp.float32),
      ],
  )(x, w1l)


def _tc2_body(pa_ref, pb_ref, x_ref, w1r_ref, b1l_ref, w2l_ref, b2l_ref,
              w2r_ref, t2_ref, hr_ref, invd_ref):
  sa = pa_ref[0] + pa_ref[1]
  sb = pb_ref[0] + pb_ref[1]
  deg = jnp.maximum(sa[:, HALF:HALF + 1], 1.0)
  invd = 1.0 / deg
  agg = jnp.concatenate([sa[:, :HALF], sb], axis=1) * invd
  h = agg + b1l_ref[...] + _dot_t(x_ref[...], w1r_ref[...])
  h = jnp.maximum(h, 0.0)
  t2_ref[...] = _dot_t(h, w2l_ref[...])
  hr_ref[...] = _dot_t(h, w2r_ref[...]) + b2l_ref[...]
  invd_ref[...] = jnp.broadcast_to(invd, (BLK, N_CLASSES))


def _tc2(pa, pb, x, w1r, b1l2, w2l, b2l2, w2r):
  return pl.pallas_call(
      _tc2_body,
      grid=(N_NODES // BLK,),
      in_specs=[
          pl.BlockSpec((NC, BLK, D_A1), lambda i: (0, i, 0)),
          pl.BlockSpec((NC, BLK, D_A2), lambda i: (0, i, 0)),
          pl.BlockSpec((BLK, D_FEAT), lambda i: (i, 0)),
          pl.BlockSpec((HIDDEN, D_FEAT), lambda i: (0, 0)),
          pl.BlockSpec((1, HIDDEN), lambda i: (0, 0)),
          pl.BlockSpec((N_CLASSES, HIDDEN), lambda i: (0, 0)),
          pl.BlockSpec((1, N_CLASSES), lambda i: (0, 0)),
          pl.BlockSpec((N_CLASSES, HIDDEN), lambda i: (0, 0)),
      ],
      out_specs=[
          pl.BlockSpec((BLK, N_CLASSES), lambda i: (i, 0)),
          pl.BlockSpec((BLK, N_CLASSES), lambda i: (i, 0)),
          pl.BlockSpec((BLK, N_CLASSES), lambda i: (i, 0)),
      ],
      out_shape=[
          jax.ShapeDtypeStruct((N_NODES, N_CLASSES), jnp.float32),
          jax.ShapeDtypeStruct((N_NODES, N_CLASSES), jnp.float32),
          jax.ShapeDtypeStruct((N_NODES, N_CLASSES), jnp.float32),
      ],
  )(pa, pb, x, w1r, b1l2, w2l, b2l2, w2r)


def _tc3_body(q_ref, invd_ref, hr_ref, o_ref):
  o_ref[...] = (q_ref[0] + q_ref[1]) * invd_ref[...] + hr_ref[...]


def _tc3(q, invd, hr):
  return pl.pallas_call(
      _tc3_body,
      grid=(N_NODES // BLK,),
      in_specs=[
          pl.BlockSpec((NC, BLK, N_CLASSES), lambda i: (0, i, 0)),
          pl.BlockSpec((BLK, N_CLASSES), lambda i: (i, 0)),
          pl.BlockSpec((BLK, N_CLASSES), lambda i: (i, 0)),
      ],
      out_specs=pl.BlockSpec((BLK, N_CLASSES), lambda i: (i, 0)),
      out_shape=jax.ShapeDtypeStruct((N_NODES, N_CLASSES), jnp.float32),
  )(q, invd, hr)


def kernel(x, edge_index, W1l, b1l, W1r, W2l, b2l, W2r):
  src = edge_index[0].astype(jnp.int32).reshape(NW * NCHUNK, CH)
  dst = edge_index[1].astype(jnp.int32).reshape(NW * NCHUNK, CH)
  b1l2 = b1l.reshape(1, HIDDEN)
  b2l2 = b2l.reshape(1, N_CLASSES)

  t1a, t1b = _tc1(x, W1l)
  pa = _make_sc_scatter_pass(D_A1)(t1a, src, dst)
  pb = _make_sc_scatter_pass(D_A2)(t1b, src, dst)
  t2, hr, invd = _tc2(pa, pb, x, W1r, b1l2, W2l, b2l2, W2r)
  q = _make_sc_scatter_pass(N_CLASSES)(t2, src, dst)
  return _tc3(q, invd, hr)
